# baseline (speedup 1.0000x reference)
import functools
import math

import numpy as np
import jax
import jax.numpy as jnp
from jax import lax
from jax.experimental import pallas as pl
from jax.experimental.pallas import tpu as pltpu
from jax.experimental.pallas import tpu_sc as plsc

D_MODEL = 768
MAX_SEQ_LEN = 2048
_SCALE = math.sqrt(float(D_MODEL))
_LANES = 16


def _pe_host() -> np.ndarray:
    pos = np.arange(MAX_SEQ_LEN, dtype=np.float64).reshape(-1, 1)
    i = np.arange(D_MODEL, dtype=np.float64)
    rads = pos / np.power(10000.0, 2.0 * np.floor(i / 2.0) / D_MODEL)
    pe = np.zeros((MAX_SEQ_LEN, D_MODEL), dtype=np.float32)
    pe[:, 0::2] = np.sin(rads[:, 0::2]).astype(np.float32)
    pe[:, 1::2] = np.cos(rads[:, 1::2]).astype(np.float32)
    return pe


_PE = _pe_host()


@functools.lru_cache(maxsize=None)
def _build(batch: int, seq: int):
    info = plsc.get_sparse_core_info()
    nc, ns = info.num_cores, info.num_subcores
    nw = nc * ns
    ppw = seq // nw
    pchunk = 32
    nchunk = ppw // pchunk
    rows = batch * pchunk
    groups = D_MODEL // _LANES

    mesh = plsc.VectorSubcoreMesh(core_axis_name="c", subcore_axis_name="s")

    @functools.partial(
        pl.kernel,
        mesh=mesh,
        out_type=jax.ShapeDtypeStruct((batch, seq, D_MODEL), jnp.float32),
        scratch_types=[
            pltpu.VMEM((nchunk, rows), jnp.int32),
            pltpu.VMEM((rows, D_MODEL), jnp.float32),
            pltpu.VMEM((pchunk, D_MODEL), jnp.float32),
            pltpu.SemaphoreType.DMA,
            pltpu.SemaphoreType.DMA,
            pltpu.SemaphoreType.DMA,
        ],
    )
    def emb(x_hbm, table_hbm, pe_hbm, out_hbm, idx_v, tok_v, pe_v,
            sem_g, sem_p, sem_o):
        wid = lax.axis_index("s") * nc + lax.axis_index("c")
        pos0 = wid * ppw
        pltpu.sync_copy(x_hbm.at[wid], idx_v)
        for c in range(nchunk):
            pos = pos0 + c * pchunk
            g = pltpu.async_copy(table_hbm.at[idx_v.at[c]], tok_v, sem_g)
            p = pltpu.async_copy(pe_hbm.at[pl.ds(pos, pchunk)], pe_v, sem_p)
            g.wait()
            p.wait()

            def row_body(r, _):
                for gi in range(groups):
                    sl = pl.ds(gi * _LANES, _LANES)
                    vp = pe_v[r, sl]
                    for b in range(batch):
                        tok_v[b * pchunk + r, sl] = (
                            tok_v[b * pchunk + r, sl] * _SCALE + vp)
                return 0

            lax.fori_loop(0, pchunk, row_body, 0)
            outs = [
                pltpu.async_copy(
                    tok_v.at[pl.ds(b * pchunk, pchunk)],
                    out_hbm.at[b, pl.ds(pos, pchunk)], sem_o)
                for b in range(batch)
            ]
            for o in outs:
                o.wait()

    return emb, nw, nchunk, pchunk


def kernel(x, table):
    b, s = x.shape
    emb, nw, nchunk, pchunk = _build(b, s)
    x3 = (x.astype(jnp.int32)
          .reshape(b, nw, nchunk, pchunk)
          .transpose(1, 2, 0, 3)
          .reshape(nw, nchunk, b * pchunk))
    pe = jnp.asarray(_PE)
    out = emb(x3, table, pe)
    return out


# speedup vs baseline: 1.2051x; 1.2051x over previous
import functools
import math

import numpy as np
import jax
import jax.numpy as jnp
from jax import lax
from jax.experimental import pallas as pl
from jax.experimental.pallas import tpu as pltpu
from jax.experimental.pallas import tpu_sc as plsc

D_MODEL = 768
MAX_SEQ_LEN = 2048
_SCALE = math.sqrt(float(D_MODEL))
_LANES = 16


def _pe_host() -> np.ndarray:
    pos = np.arange(MAX_SEQ_LEN, dtype=np.float64).reshape(-1, 1)
    i = np.arange(D_MODEL, dtype=np.float64)
    rads = pos / np.power(10000.0, 2.0 * np.floor(i / 2.0) / D_MODEL)
    pe = np.zeros((MAX_SEQ_LEN, D_MODEL), dtype=np.float32)
    pe[:, 0::2] = np.sin(rads[:, 0::2]).astype(np.float32)
    pe[:, 1::2] = np.cos(rads[:, 1::2]).astype(np.float32)
    return pe


_PE = _pe_host()


@functools.lru_cache(maxsize=None)
def _build(batch: int, seq: int):
    info = plsc.get_sparse_core_info()
    nc, ns = info.num_cores, info.num_subcores
    nw = nc * ns
    rpw = batch * seq // nw
    wpb = nw // batch
    chunk = 64
    nchunk = rpw // chunk
    groups = D_MODEL // _LANES

    mesh = plsc.VectorSubcoreMesh(core_axis_name="c", subcore_axis_name="s")

    @functools.partial(
        pl.kernel,
        mesh=mesh,
        out_type=jax.ShapeDtypeStruct((batch, seq, D_MODEL), jnp.float32),
        scratch_types=[
            pltpu.VMEM((rpw,), jnp.int32),
            pltpu.VMEM((chunk, D_MODEL), jnp.float32),
            pltpu.VMEM((chunk, D_MODEL), jnp.float32),
            pltpu.SemaphoreType.DMA,
            pltpu.SemaphoreType.DMA,
            pltpu.SemaphoreType.DMA,
        ],
    )
    def emb(x_hbm, table_hbm, pe_hbm, out_hbm, idx_v, tok_v, pe_v,
            sem_g, sem_p, sem_o):
        wid = lax.axis_index("s") * nc + lax.axis_index("c")
        bi = wid // wpb
        seq0 = (wid % wpb) * rpw
        pltpu.sync_copy(x_hbm.at[bi, pl.ds(seq0, rpw)], idx_v)
        for c in range(nchunk):
            g = pltpu.async_copy(
                table_hbm.at[idx_v.at[pl.ds(c * chunk, chunk)]], tok_v, sem_g)
            p = pltpu.async_copy(
                pe_hbm.at[pl.ds(seq0 + c * chunk, chunk)], pe_v, sem_p)
            g.wait()
            p.wait()

            @plsc.parallel_loop(0, chunk, unroll=2)
            def _row(r):
                for gi in range(groups):
                    sl = pl.ds(gi * _LANES, _LANES)
                    tok_v[r, sl] = tok_v[r, sl] * _SCALE + pe_v[r, sl]

            pltpu.async_copy(
                tok_v, out_hbm.at[bi, pl.ds(seq0 + c * chunk, chunk)],
                sem_o).wait()

    return emb


def kernel(x, table):
    b, s = x.shape
    emb = _build(b, s)
    pe = jnp.asarray(_PE)
    return emb(x, table, pe)
